# baseline (speedup 1.0000x reference)
import functools

import jax
import jax.numpy as jnp
from jax import lax
from jax.experimental import pallas as pl
from jax.experimental.pallas import tpu as pltpu
from jax.experimental.pallas import tpu_sc as plsc

B = 16384
E = 128
BN_EPS = 1e-3
NC, NS = 2, 16
NW = NC * NS
BPW = B // NW
CHUNK = 128
NCH = BPW // CHUNK
LANES = 16
TB = 8192


def _sc_body(u_hbm, c_hbm, umf_hbm, cmf_hbm, unn_hbm, cnn_hbm, ubt_hbm,
             cbt_hbm, uv_out, cv_out, ubcb_out, part_out,
             idx_u, idx_c, amf, bmf, ann, bnn, bu, bc, stage, svec,
             s_in0, s_in1, s_out0, s_out1):
    wid = lax.axis_index("s") * NC + lax.axis_index("c")
    base = wid * BPW
    s_in = (s_in0, s_in1)
    s_out = (s_out0, s_out1)

    for j in range(NCH):
        pltpu.sync_copy(u_hbm.at[pl.ds(base + j * CHUNK, CHUNK)], idx_u.at[j])
        pltpu.sync_copy(c_hbm.at[pl.ds(base + j * CHUNK, CHUNK)], idx_c.at[j])

    acc = jnp.zeros((LANES,), jnp.float32)
    for j in range(NCH):
        p = j % 2
        off = base + j * CHUNK
        cp_ann = pltpu.async_copy(unn_hbm.at[idx_u.at[j]], ann.at[0], s_in[0])
        cp_bnn = pltpu.async_copy(cnn_hbm.at[idx_c.at[j]], bnn.at[0], s_in[0])
        cp_amf = pltpu.async_copy(umf_hbm.at[idx_u.at[j]], amf.at[0], s_in[0])
        cp_bmf = pltpu.async_copy(cmf_hbm.at[idx_c.at[j]], bmf.at[0], s_in[0])
        cp_bu = pltpu.async_copy(ubt_hbm.at[idx_u.at[j]], bu.at[0], s_in[1])
        cp_bc = pltpu.async_copy(cbt_hbm.at[idx_c.at[j]], bc.at[0], s_in[1])

        cp_ann.wait()
        pltpu.sync_copy(ann.at[0], uv_out.at[pl.ds(off, CHUNK)])
        cp_bnn.wait()
        pltpu.sync_copy(bnn.at[0], cv_out.at[pl.ds(off, CHUNK)])
        cp_bu.wait()
        cp_bc.wait()
        for k in range(CHUNK // LANES):
            sl = pl.ds(k * LANES, LANES)
            stage[0, sl] = bu[0, sl] + bc[0, sl]
        pltpu.sync_copy(stage.at[0], ubcb_out.at[pl.ds(off, CHUNK)])

        cp_amf.wait()
        cp_bmf.wait()

        def row_body(i, a):
            for k in range(E // LANES):
                sl = pl.ds(k * LANES, LANES)
                a = a + amf[0, i, sl] * bmf[0, i, sl]
            return a

        acc = lax.fori_loop(0, CHUNK, row_body, acc)

    svec[...] = acc
    pltpu.sync_copy(svec, part_out.at[pl.ds(wid * LANES, LANES)])


def _sc_gather(u, c, umf, cmf, unn, cnn, ubt, cbt):
    mesh = plsc.VectorSubcoreMesh(core_axis_name="c", subcore_axis_name="s")
    f32 = jnp.float32
    kern = functools.partial(
        pl.kernel,
        mesh=mesh,
        out_type=[
            jax.ShapeDtypeStruct((B, E), f32),
            jax.ShapeDtypeStruct((B, E), f32),
            jax.ShapeDtypeStruct((B,), f32),
            jax.ShapeDtypeStruct((NW * LANES,), f32),
        ],
        scratch_types=[
            pltpu.VMEM((NCH, CHUNK), jnp.int32),
            pltpu.VMEM((NCH, CHUNK), jnp.int32),
            pltpu.VMEM((1, CHUNK, E), f32),
            pltpu.VMEM((1, CHUNK, E), f32),
            pltpu.VMEM((1, CHUNK, E), f32),
            pltpu.VMEM((1, CHUNK, E), f32),
            pltpu.VMEM((1, CHUNK), f32),
            pltpu.VMEM((1, CHUNK), f32),
            pltpu.VMEM((1, CHUNK), f32),
            pltpu.VMEM((LANES,), f32),
            pltpu.SemaphoreType.DMA,
            pltpu.SemaphoreType.DMA,
            pltpu.SemaphoreType.DMA,
            pltpu.SemaphoreType.DMA,
        ],
    )(_sc_body)
    return kern(u, c, umf, cmf, unn, cnn, ubt, cbt)


def _tc_body(uv, cv, ubcb, part, w1u, w1c, b1, g1, be1, m1, v1,
             w2, b2, g2, be2, m2, v2, wv, wmf, bout, out):
    s1 = g1[...] * lax.rsqrt(v1[...] + BN_EPS)
    t1 = be1[...] - m1[...] * s1
    s2 = g2[...] * lax.rsqrt(v2[...] + BN_EPS)
    t2 = be2[...] - m2[...] * s2
    x1 = (jnp.dot(uv[...], w1u[...], preferred_element_type=jnp.float32)
          + jnp.dot(cv[...], w1c[...], preferred_element_type=jnp.float32)
          + b1[...])
    x1 = jnp.maximum(x1, 0.0) * s1 + t1
    x2 = jnp.dot(x1, w2[...], preferred_element_type=jnp.float32) + b2[...]
    x2 = jnp.maximum(x2, 0.0) * s2 + t2
    dot_mf = jnp.sum(part[...])
    z = ((dot_mf + ubcb[...].reshape(TB, 1)) * wmf[0, 0]
         + jnp.dot(x2, wv[...], preferred_element_type=jnp.float32)
         + bout[0, 0])
    out[...] = 1.0 / (1.0 + jnp.exp(-z))


def _tc_mlp(uv, cv, ubcb, part, w1u, w1c, b1, g1, be1, m1, v1,
            w2, b2, g2, be2, m2, v2, wv, wmf, bout):
    grid = (B // TB,)
    row = lambda i: (i, 0)
    rep = lambda i: (0, 0)
    return pl.pallas_call(
        _tc_body,
        grid=grid,
        in_specs=[
            pl.BlockSpec((TB, E), row),
            pl.BlockSpec((TB, E), row),
            pl.BlockSpec((TB,), lambda i: (i,)),
            pl.BlockSpec((1, NW * LANES), rep),
            pl.BlockSpec((E, 64), rep),
            pl.BlockSpec((E, 64), rep),
            pl.BlockSpec((1, 64), rep),
            pl.BlockSpec((1, 64), rep),
            pl.BlockSpec((1, 64), rep),
            pl.BlockSpec((1, 64), rep),
            pl.BlockSpec((1, 64), rep),
            pl.BlockSpec((64, 32), rep),
            pl.BlockSpec((1, 32), rep),
            pl.BlockSpec((1, 32), rep),
            pl.BlockSpec((1, 32), rep),
            pl.BlockSpec((1, 32), rep),
            pl.BlockSpec((1, 32), rep),
            pl.BlockSpec((32, 1), rep),
            pl.BlockSpec((1, 1), rep),
            pl.BlockSpec((1, 1), rep),
        ],
        out_specs=pl.BlockSpec((TB, 1), row),
        out_shape=jax.ShapeDtypeStruct((B, 1), jnp.float32),
    )(uv, cv, ubcb, part, w1u, w1c, b1, g1, be1, m1, v1,
      w2, b2, g2, be2, m2, v2, wv, wmf, bout)


def kernel(inputs, user_emb_mf, user_bias_mf, course_emb_mf, course_bias_mf,
           user_emb_nn, course_emb_nn, W1, b1, gamma1, beta1, mean1, var1,
           W2, b2, gamma2, beta2, mean2, var2, Wout, bout):
    uv, cv, ubcb, part = _sc_gather(inputs[:, 0], inputs[:, 1], user_emb_mf,
                                    course_emb_mf,
                                    user_emb_nn, course_emb_nn,
                                    user_bias_mf[:, 0], course_bias_mf[:, 0])
    r1 = lambda a: a.reshape(1, -1)
    return _tc_mlp(uv, cv, ubcb, part.reshape(1, NW * LANES),
                   W1[:E], W1[E:], r1(b1), r1(gamma1), r1(beta1), r1(mean1),
                   r1(var1), W2, r1(b2), r1(gamma2), r1(beta2), r1(mean2),
                   r1(var2), Wout[1:], Wout[:1], bout.reshape(1, 1))


# speedup vs baseline: 1.0132x; 1.0132x over previous
import functools

import jax
import jax.numpy as jnp
from jax import lax
from jax.experimental import pallas as pl
from jax.experimental.pallas import tpu as pltpu
from jax.experimental.pallas import tpu_sc as plsc

B = 16384
E = 128
BN_EPS = 1e-3
NC, NS = 2, 16
NW = NC * NS
BPW = B // NW
CHUNK = 128
NCH = BPW // CHUNK
LANES = 16
TB = 4096


def _sc_body(u_hbm, c_hbm, umf_hbm, cmf_hbm, unn_hbm, cnn_hbm, ubt_hbm,
             cbt_hbm, uv_out, cv_out, ubcb_out, part_out,
             idx_u, idx_c, amf, bmf, ann, bnn, bu, bc, stage, svec,
             s_in0, s_in1, s_out0, s_out1):
    wid = lax.axis_index("s") * NC + lax.axis_index("c")
    base = wid * BPW
    s_in = (s_in0, s_in1)
    s_out = (s_out0, s_out1)

    for j in range(NCH):
        pltpu.sync_copy(u_hbm.at[pl.ds(base + j * CHUNK, CHUNK)], idx_u.at[j])
        pltpu.sync_copy(c_hbm.at[pl.ds(base + j * CHUNK, CHUNK)], idx_c.at[j])

    acc = jnp.zeros((LANES,), jnp.float32)
    for j in range(NCH):
        p = j % 2
        off = base + j * CHUNK
        cp_ann = pltpu.async_copy(unn_hbm.at[idx_u.at[j]], ann.at[0], s_in[0])
        cp_bnn = pltpu.async_copy(cnn_hbm.at[idx_c.at[j]], bnn.at[0], s_in[0])
        cp_amf = pltpu.async_copy(umf_hbm.at[idx_u.at[j]], amf.at[0], s_in[0])
        cp_bmf = pltpu.async_copy(cmf_hbm.at[idx_c.at[j]], bmf.at[0], s_in[0])
        cp_bu = pltpu.async_copy(ubt_hbm.at[idx_u.at[j]], bu.at[0], s_in[1])
        cp_bc = pltpu.async_copy(cbt_hbm.at[idx_c.at[j]], bc.at[0], s_in[1])

        cp_ann.wait()
        pltpu.sync_copy(ann.at[0], uv_out.at[pl.ds(off, CHUNK)])
        cp_bnn.wait()
        pltpu.sync_copy(bnn.at[0], cv_out.at[pl.ds(off, CHUNK)])
        cp_bu.wait()
        cp_bc.wait()
        for k in range(CHUNK // LANES):
            sl = pl.ds(k * LANES, LANES)
            stage[0, sl] = bu[0, sl] + bc[0, sl]
        pltpu.sync_copy(stage.at[0], ubcb_out.at[pl.ds(off, CHUNK)])

        cp_amf.wait()
        cp_bmf.wait()

        def row_body(i, a):
            for k in range(E // LANES):
                sl = pl.ds(k * LANES, LANES)
                a = a + amf[0, i, sl] * bmf[0, i, sl]
            return a

        acc = lax.fori_loop(0, CHUNK, row_body, acc)

    svec[...] = acc
    pltpu.sync_copy(svec, part_out.at[pl.ds(wid * LANES, LANES)])


def _sc_gather(u, c, umf, cmf, unn, cnn, ubt, cbt):
    mesh = plsc.VectorSubcoreMesh(core_axis_name="c", subcore_axis_name="s")
    f32 = jnp.float32
    kern = functools.partial(
        pl.kernel,
        mesh=mesh,
        out_type=[
            jax.ShapeDtypeStruct((B, E), f32),
            jax.ShapeDtypeStruct((B, E), f32),
            jax.ShapeDtypeStruct((B,), f32),
            jax.ShapeDtypeStruct((NW * LANES,), f32),
        ],
        scratch_types=[
            pltpu.VMEM((NCH, CHUNK), jnp.int32),
            pltpu.VMEM((NCH, CHUNK), jnp.int32),
            pltpu.VMEM((1, CHUNK, E), f32),
            pltpu.VMEM((1, CHUNK, E), f32),
            pltpu.VMEM((1, CHUNK, E), f32),
            pltpu.VMEM((1, CHUNK, E), f32),
            pltpu.VMEM((1, CHUNK), f32),
            pltpu.VMEM((1, CHUNK), f32),
            pltpu.VMEM((1, CHUNK), f32),
            pltpu.VMEM((LANES,), f32),
            pltpu.SemaphoreType.DMA,
            pltpu.SemaphoreType.DMA,
            pltpu.SemaphoreType.DMA,
            pltpu.SemaphoreType.DMA,
        ],
    )(_sc_body)
    return kern(u, c, umf, cmf, unn, cnn, ubt, cbt)


def _tc_body(uv, cv, ubcb, part, w1u, w1c, b1, g1, be1, m1, v1,
             w2, b2, g2, be2, m2, v2, wv, wmf, bout, out):
    s1 = g1[...] * lax.rsqrt(v1[...] + BN_EPS)
    t1 = be1[...] - m1[...] * s1
    s2 = g2[...] * lax.rsqrt(v2[...] + BN_EPS)
    t2 = be2[...] - m2[...] * s2
    x1 = (jnp.dot(uv[...], w1u[...], preferred_element_type=jnp.float32)
          + jnp.dot(cv[...], w1c[...], preferred_element_type=jnp.float32)
          + b1[...])
    x1 = jnp.maximum(x1, 0.0) * s1 + t1
    x2 = jnp.dot(x1, w2[...], preferred_element_type=jnp.float32) + b2[...]
    x2 = jnp.maximum(x2, 0.0) * s2 + t2
    dot_mf = jnp.sum(part[...])
    z = ((dot_mf + ubcb[...].reshape(TB, 1)) * wmf[0, 0]
         + jnp.dot(x2, wv[...], preferred_element_type=jnp.float32)
         + bout[0, 0])
    out[...] = 1.0 / (1.0 + jnp.exp(-z))


def _tc_mlp(uv, cv, ubcb, part, w1u, w1c, b1, g1, be1, m1, v1,
            w2, b2, g2, be2, m2, v2, wv, wmf, bout):
    grid = (B // TB,)
    row = lambda i: (i, 0)
    rep = lambda i: (0, 0)
    return pl.pallas_call(
        _tc_body,
        grid=grid,
        in_specs=[
            pl.BlockSpec((TB, E), row),
            pl.BlockSpec((TB, E), row),
            pl.BlockSpec((TB,), lambda i: (i,)),
            pl.BlockSpec((1, NW * LANES), rep),
            pl.BlockSpec((E, 64), lambda i: (0, 0)),
            pl.BlockSpec((E, 64), lambda i: (1, 0)),
            pl.BlockSpec((1, 64), rep),
            pl.BlockSpec((1, 64), rep),
            pl.BlockSpec((1, 64), rep),
            pl.BlockSpec((1, 64), rep),
            pl.BlockSpec((1, 64), rep),
            pl.BlockSpec((64, 32), rep),
            pl.BlockSpec((1, 32), rep),
            pl.BlockSpec((1, 32), rep),
            pl.BlockSpec((1, 32), rep),
            pl.BlockSpec((1, 32), rep),
            pl.BlockSpec((1, 32), rep),
            pl.BlockSpec((32, 1), rep),
            pl.BlockSpec((1, 1), rep),
            pl.BlockSpec((1, 1), rep),
        ],
        out_specs=pl.BlockSpec((TB, 1), row),
        out_shape=jax.ShapeDtypeStruct((B, 1), jnp.float32),
    )(uv, cv, ubcb, part, w1u, w1c, b1, g1, be1, m1, v1,
      w2, b2, g2, be2, m2, v2, wv, wmf, bout)


def kernel(inputs, user_emb_mf, user_bias_mf, course_emb_mf, course_bias_mf,
           user_emb_nn, course_emb_nn, W1, b1, gamma1, beta1, mean1, var1,
           W2, b2, gamma2, beta2, mean2, var2, Wout, bout):
    uv, cv, ubcb, part = _sc_gather(inputs[:, 0], inputs[:, 1], user_emb_mf,
                                    course_emb_mf,
                                    user_emb_nn, course_emb_nn,
                                    user_bias_mf[:, 0], course_bias_mf[:, 0])
    r1 = lambda a: a.reshape(1, -1)
    return _tc_mlp(uv, cv, ubcb, part.reshape(1, NW * LANES),
                   W1, W1, r1(b1), r1(gamma1), r1(beta1), r1(mean1),
                   r1(var1), W2, r1(b2), r1(gamma2), r1(beta2), r1(mean2),
                   r1(var2), Wout[1:], Wout[:1], bout.reshape(1, 1))
